# baseline (speedup 1.0000x reference)
import jax
import jax.numpy as jnp
from jax.experimental import pallas as pl

_LB = 8


def _fused_body(x_ref, cal_ref, w_ref, posb_ref, o_ref):
    iota = jax.lax.broadcasted_iota(jnp.int32, (24, 1), 0)
    B = o_ref.shape[0]
    for j in range(_LB):
        di = cal_ref[j : j + 1, 0:B]
        mi = cal_ref[j : j + 1, B : 2 * B]
        onehot_t = ((di == iota) | ((mi + 8) == iota)).astype(jnp.float32)
        xa = jnp.concatenate([x_ref[j], onehot_t], axis=0)
        res = jax.lax.dot_general(
            w_ref[...],
            xa,
            dimension_numbers=(((0,), (0,)), ((), ())),
            preferred_element_type=jnp.float32,
        )
        res = res + posb_ref[0, :, j : j + 1]
        o_ref[:, j, :] = jnp.transpose(res)


@jax.jit
def kernel(x, calendar_features, W_feat, b_feat, pos_table, day_table, month_table):
    B, L, F = x.shape
    D = W_feat.shape[1]
    xT = jnp.transpose(x, (1, 2, 0))
    calT = jnp.transpose(calendar_features.astype(jnp.int32), (1, 2, 0)).reshape(
        L, 2 * B
    )
    tab = jnp.zeros((24, D), dtype=jnp.float32)
    tab = tab.at[0:7].set(day_table)
    tab = tab.at[8:20].set(month_table)
    wc = jnp.concatenate([W_feat, tab], axis=0)
    posb = pos_table[:L].T + b_feat[:, None]
    posb3 = posb.reshape(D, L // _LB, _LB).transpose(1, 0, 2)

    return pl.pallas_call(
        _fused_body,
        grid=(L // _LB,),
        in_specs=[
            pl.BlockSpec((_LB, F, B), lambda i: (i, 0, 0)),
            pl.BlockSpec((_LB, 2 * B), lambda i: (i, 0)),
            pl.BlockSpec((F + 24, D), lambda i: (0, 0)),
            pl.BlockSpec((1, D, _LB), lambda i: (i, 0, 0)),
        ],
        out_specs=pl.BlockSpec((B, _LB, D), lambda i: (0, i, 0)),
        out_shape=jax.ShapeDtypeStruct((B, L, D), jnp.float32),
    )(xT, calT, wc, posb3)


# speedup vs baseline: 1.0169x; 1.0169x over previous
import jax
import jax.numpy as jnp
from jax.experimental import pallas as pl

_LB = 8


def _fused_body(x_ref, cal_ref, w_ref, posb_ref, o_ref):
    iota = jax.lax.broadcasted_iota(jnp.int32, (24, 1), 0)
    for j in range(_LB):
        di = cal_ref[j, 0:1, :]
        mi = cal_ref[j, 1:2, :]
        onehot_t = ((di == iota) | ((mi + 8) == iota)).astype(jnp.float32)
        xa = jnp.concatenate([x_ref[j], onehot_t], axis=0)
        res = jax.lax.dot_general(
            w_ref[...],
            xa,
            dimension_numbers=(((0,), (0,)), ((), ())),
            preferred_element_type=jnp.float32,
        )
        res = res + posb_ref[0, :, j : j + 1]
        o_ref[:, j, :] = jnp.transpose(res)


@jax.jit
def kernel(x, calendar_features, W_feat, b_feat, pos_table, day_table, month_table):
    B, L, F = x.shape
    D = W_feat.shape[1]
    xT = jnp.transpose(x, (1, 2, 0))
    calT = jnp.transpose(calendar_features.astype(jnp.int32), (1, 2, 0))
    tab = jnp.zeros((24, D), dtype=jnp.float32)
    tab = tab.at[0:7].set(day_table)
    tab = tab.at[8:20].set(month_table)
    wc = jnp.concatenate([W_feat, tab], axis=0)
    posb = pos_table[:L].T + b_feat[:, None]
    posb3 = posb.reshape(D, L // _LB, _LB).transpose(1, 0, 2)

    return pl.pallas_call(
        _fused_body,
        grid=(L // _LB,),
        in_specs=[
            pl.BlockSpec((_LB, F, B), lambda i: (i, 0, 0)),
            pl.BlockSpec((_LB, 2, B), lambda i: (i, 0, 0)),
            pl.BlockSpec((F + 24, D), lambda i: (0, 0)),
            pl.BlockSpec((1, D, _LB), lambda i: (i, 0, 0)),
        ],
        out_specs=pl.BlockSpec((B, _LB, D), lambda i: (0, i, 0)),
        out_shape=jax.ShapeDtypeStruct((B, L, D), jnp.float32),
    )(xT, calT, wc, posb3)
